# baseline (speedup 1.0000x reference)
import jax
import jax.numpy as jnp
from jax.experimental import pallas as pl

N = 100000
D = 256
BLOCK_ROWS = 4000
GRID = N // BLOCK_ROWS


def _body(u_ref, v_ref, o_ref):
    prod = u_ref[...] * v_ref[...]
    ones = jnp.ones((D, 1), jnp.float32)
    s = jax.lax.dot_general(
        prod, ones, (((1,), (0,)), ((), ())),
        precision=jax.lax.Precision.HIGHEST,
        preferred_element_type=jnp.float32,
    )
    o_ref[...] = s.reshape(1, 1, BLOCK_ROWS)


def kernel(gu, gi):
    out3 = pl.pallas_call(
        _body,
        grid=(GRID,),
        in_specs=[
            pl.BlockSpec((BLOCK_ROWS, D), lambda i: (i, 0)),
            pl.BlockSpec((BLOCK_ROWS, D), lambda i: (i, 0)),
        ],
        out_specs=pl.BlockSpec((1, 1, BLOCK_ROWS), lambda i: (i, 0, 0)),
        out_shape=jax.ShapeDtypeStruct((GRID, 1, BLOCK_ROWS), jnp.float32),
    )(gu, gi)
    return out3.reshape(N)


# speedup vs baseline: 1.6639x; 1.6639x over previous
import jax
import jax.numpy as jnp
from jax.experimental import pallas as pl

N = 100000
D = 256
BLOCK_ROWS = 4000
GRID = N // BLOCK_ROWS


def _body(u_ref, v_ref, o_ref):
    prod = u_ref[...] * v_ref[...]
    ones = jnp.ones((1, D), jnp.float32)
    s = jax.lax.dot_general(
        ones, prod, (((1,), (1,)), ((), ())),
        preferred_element_type=jnp.float32,
    )
    o_ref[...] = s.reshape(1, 1, BLOCK_ROWS)


def kernel(gu, gi):
    out3 = pl.pallas_call(
        _body,
        grid=(GRID,),
        in_specs=[
            pl.BlockSpec((BLOCK_ROWS, D), lambda i: (i, 0)),
            pl.BlockSpec((BLOCK_ROWS, D), lambda i: (i, 0)),
        ],
        out_specs=pl.BlockSpec((1, 1, BLOCK_ROWS), lambda i: (i, 0, 0)),
        out_shape=jax.ShapeDtypeStruct((GRID, 1, BLOCK_ROWS), jnp.float32),
    )(gu, gi)
    return out3.reshape(N)
